# baseline (speedup 1.0000x reference)
import functools

import jax
import jax.numpy as jnp
from jax import lax
from jax.experimental import pallas as pl
from jax.experimental.pallas import tpu as pltpu
from jax.experimental.pallas import tpu_sc as plsc

N = 10000
F = 128
NPAD = 10240
E = 320000
CH = 128
NCH = 80
IB = 16
TILES = 32
EPAD = TILES * NCH * CH
RPT = NPAD // 16
BLK = 512

_MESH = dict(core_axis_name="c", subcore_axis_name="s")


def _sc_degree(dst2d, zeros1d):
    @functools.partial(
        pl.kernel,
        out_type=jax.ShapeDtypeStruct((2, NPAD), jnp.float32),
        mesh=plsc.VectorSubcoreMesh(**_MESH),
        scratch_types=[
            pltpu.VMEM((NCH, CH), jnp.int32),
            pltpu.VMEM((CH,), jnp.float32),
            pltpu.VMEM_SHARED((NPAD,), jnp.float32),
        ],
    )
    def k(dst_hbm, z_hbm, out_hbm, dst_v, ones_v, acc):
        c = lax.axis_index("c")
        s = lax.axis_index("s")
        t = c * 16 + s
        pltpu.sync_copy(dst_hbm.at[pl.ds(t * NCH, NCH)], dst_v)
        for i in range(CH // 16):
            ones_v[pl.ds(i * 16, 16)] = jnp.ones((16,), jnp.float32)
        pltpu.sync_copy(z_hbm.at[pl.ds(s * RPT, RPT)], acc.at[pl.ds(s * RPT, RPT)])
        plsc.subcore_barrier()

        @pl.loop(0, NCH)
        def _(j):
            pltpu.sync_copy(ones_v, acc.at[dst_v.at[j]], add=True)

        plsc.subcore_barrier()
        pltpu.sync_copy(acc.at[pl.ds(s * RPT, RPT)], out_hbm.at[c, pl.ds(s * RPT, RPT)])

    return k(dst2d, zeros1d)


def _sc_scatter_rows(g, src2d, dst2d, zeros2d):
    @functools.partial(
        pl.kernel,
        out_type=jax.ShapeDtypeStruct((2, NPAD, F), jnp.float32),
        mesh=plsc.VectorSubcoreMesh(**_MESH),
        scratch_types=[
            pltpu.VMEM((IB, CH), jnp.int32),
            pltpu.VMEM((IB, CH), jnp.int32),
            pltpu.VMEM((2, CH, F), jnp.float32),
            pltpu.VMEM_SHARED((NPAD, F), jnp.float32),
            pltpu.SemaphoreType.DMA,
            pltpu.SemaphoreType.DMA,
        ],
    )
    def k(g_hbm, src_hbm, dst_hbm, z_hbm, out_hbm, src_v, dst_v, rows_v, acc,
          sem0, sem1):
        c = lax.axis_index("c")
        s = lax.axis_index("s")
        t = c * 16 + s
        pltpu.sync_copy(z_hbm.at[pl.ds(s * RPT, RPT)], acc.at[pl.ds(s * RPT, RPT)])
        plsc.subcore_barrier()

        @pl.loop(0, NCH // IB)
        def _(ib):
            base = t * NCH + ib * IB
            pltpu.sync_copy(src_hbm.at[pl.ds(base, IB)], src_v)
            pltpu.sync_copy(dst_hbm.at[pl.ds(base, IB)], dst_v)

            @pl.loop(0, IB, step=2)
            def _(j):
                d0 = pltpu.async_copy(g_hbm.at[src_v.at[j]], rows_v.at[0], sem0)
                d1 = pltpu.async_copy(g_hbm.at[src_v.at[j + 1]], rows_v.at[1],
                                      sem1)
                d0.wait()
                pltpu.sync_copy(rows_v.at[0], acc.at[dst_v.at[j]], add=True)
                d1.wait()
                pltpu.sync_copy(rows_v.at[1], acc.at[dst_v.at[j + 1]], add=True)

        plsc.subcore_barrier()
        pltpu.sync_copy(acc.at[pl.ds(s * RPT, RPT)], out_hbm.at[c, pl.ds(s * RPT, RPT)])

    return k(g, src2d, dst2d, zeros2d)


def _dinv_of(d0, d1):
    return lax.rsqrt(d0[...] + d1[...] + 1.0)[:, None]


def _tc_first(x_ref, w_ref, d0, d1, o_ref):
    dinv = _dinv_of(d0, d1)
    o_ref[...] = jnp.dot(x_ref[...], w_ref[...],
                         preferred_element_type=jnp.float32) * dinv


def _tc_mid(a0, a1, g_ref, b_ref, w_ref, d0, d1, o_ref):
    dinv = _dinv_of(d0, d1)
    z = (a0[...] + a1[...] + g_ref[...]) * dinv + b_ref[...][None, :]
    h = jnp.maximum(z, 0.0) + jnp.log1p(jnp.exp(-jnp.abs(z)))
    o_ref[...] = jnp.dot(h, w_ref[...],
                         preferred_element_type=jnp.float32) * dinv


def _tc_last(a0, a1, g_ref, b_ref, wf_ref, bf_ref, d0, d1, o_ref):
    dinv = _dinv_of(d0, d1)
    z = (a0[...] + a1[...] + g_ref[...]) * dinv + b_ref[...][None, :]
    logits = jnp.dot(z, wf_ref[...],
                     preferred_element_type=jnp.float32) + bf_ref[...][None, :]
    m = jnp.max(logits, axis=1, keepdims=True)
    e = jnp.exp(logits - m)
    o_ref[...] = e / jnp.sum(e, axis=1, keepdims=True)


def _row_spec():
    return pl.BlockSpec((BLK, F), lambda i: (i, 0))


def _full_spec(shape):
    nd = len(shape)
    return pl.BlockSpec(shape, lambda i: (0,) * nd)


def _vec_spec():
    return pl.BlockSpec((BLK,), lambda i: (i,))


def _tc_call(body, in_specs, args):
    return pl.pallas_call(
        body,
        grid=(NPAD // BLK,),
        in_specs=in_specs,
        out_specs=_row_spec(),
        out_shape=jax.ShapeDtypeStruct((NPAD, F), jnp.float32),
    )(*args)


def kernel(x, edge_index, W1, b1, W2, b2, Wf, bf):
    src = edge_index[0].astype(jnp.int32)
    dst = edge_index[1].astype(jnp.int32)
    pad = EPAD - E
    pad_dst = N + (jnp.arange(pad, dtype=jnp.int32) % (NPAD - N))

    def _chunked(a):
        a2 = a.reshape(NCH, TILES, CH).transpose(1, 0, 2)
        return a2.reshape(TILES * NCH, CH)

    src_f = jnp.concatenate([src, jnp.full((pad,), N, jnp.int32)])
    dst_f = jnp.concatenate([dst, pad_dst])
    key = jnp.sort((src_f << 14) | dst_f)
    src2d = _chunked(key >> 14)
    dst2d = _chunked(key & ((1 << 14) - 1))
    xp = jnp.zeros((NPAD, F), x.dtype).at[:N].set(x)
    zeros1d = jnp.zeros((NPAD,), jnp.float32)
    zeros2d = jnp.zeros((NPAD, F), jnp.float32)

    deg = _sc_degree(dst2d, zeros1d)
    d0, d1 = deg[0], deg[1]

    g1 = _tc_call(
        _tc_first,
        [_row_spec(), _full_spec((F, F)), _vec_spec(), _vec_spec()],
        (xp, W1, d0, d1))
    acc1 = _sc_scatter_rows(g1, src2d, dst2d, zeros2d)

    g2 = _tc_call(
        _tc_mid,
        [_row_spec(), _row_spec(), _row_spec(), _full_spec((F,)),
         _full_spec((F, F)), _vec_spec(), _vec_spec()],
        (acc1[0], acc1[1], g1, b1, W2, d0, d1))
    acc2 = _sc_scatter_rows(g2, src2d, dst2d, zeros2d)

    wf_pad = jnp.zeros((F, F), jnp.float32).at[:, :7].set(Wf)
    bf_pad = jnp.full((F,), -1e30, jnp.float32).at[:7].set(bf)
    probs = _tc_call(
        _tc_last,
        [_row_spec(), _row_spec(), _row_spec(), _full_spec((F,)),
         _full_spec((F, F)), _full_spec((F,)), _vec_spec(), _vec_spec()],
        (acc2[0], acc2[1], g2, b2, wf_pad, bf_pad, d0, d1))
    return probs[:N, :7]


# speedup vs baseline: 1.4541x; 1.4541x over previous
import functools

import jax
import jax.numpy as jnp
from jax import lax
from jax.experimental import pallas as pl
from jax.experimental.pallas import tpu as pltpu
from jax.experimental.pallas import tpu_sc as plsc

N = 10000
F = 128
NPAD = 10240
E = 320000
CH = 128
NCH = 80
IB = 16
TILES = 32
EPAD = TILES * NCH * CH
RPT = NPAD // 16
BLK = 512

_MESH = dict(core_axis_name="c", subcore_axis_name="s")


def _sc_degree(dst2d, zeros1d):
    @functools.partial(
        pl.kernel,
        out_type=jax.ShapeDtypeStruct((2, NPAD), jnp.float32),
        mesh=plsc.VectorSubcoreMesh(**_MESH),
        scratch_types=[
            pltpu.VMEM((NCH, CH), jnp.int32),
            pltpu.VMEM((CH,), jnp.float32),
            pltpu.VMEM_SHARED((NPAD,), jnp.float32),
        ],
    )
    def k(dst_hbm, z_hbm, out_hbm, dst_v, ones_v, acc):
        c = lax.axis_index("c")
        s = lax.axis_index("s")
        t = c * 16 + s
        pltpu.sync_copy(dst_hbm.at[pl.ds(t * NCH, NCH)], dst_v)
        for i in range(CH // 16):
            ones_v[pl.ds(i * 16, 16)] = jnp.ones((16,), jnp.float32)
        pltpu.sync_copy(z_hbm.at[pl.ds(s * RPT, RPT)], acc.at[pl.ds(s * RPT, RPT)])
        plsc.subcore_barrier()

        @pl.loop(0, NCH)
        def _(j):
            pltpu.sync_copy(ones_v, acc.at[dst_v.at[j]], add=True)

        plsc.subcore_barrier()
        pltpu.sync_copy(acc.at[pl.ds(s * RPT, RPT)], out_hbm.at[c, pl.ds(s * RPT, RPT)])

    return k(dst2d, zeros1d)


def _sc_scatter_rows(g, src2d, dst2d, zeros2d):
    @functools.partial(
        pl.kernel,
        out_type=jax.ShapeDtypeStruct((2, NPAD, F), jnp.float32),
        mesh=plsc.VectorSubcoreMesh(**_MESH),
        scratch_types=[
            pltpu.VMEM((IB, CH), jnp.int32),
            pltpu.VMEM((IB, CH), jnp.int32),
            pltpu.VMEM((2, CH, F), jnp.float32),
            pltpu.VMEM_SHARED((NPAD, F), jnp.float32),
            [pltpu.SemaphoreType.DMA] * 2,
            [pltpu.SemaphoreType.DMA] * 2,
        ],
    )
    def k(g_hbm, src_hbm, dst_hbm, z_hbm, out_hbm, src_v, dst_v, rows_v, acc,
          gsems, ssems):
        c = lax.axis_index("c")
        s = lax.axis_index("s")
        t = c * 16 + s
        pltpu.sync_copy(z_hbm.at[pl.ds(s * RPT, RPT)], acc.at[pl.ds(s * RPT, RPT)])
        plsc.subcore_barrier()

        def _wait_scat(b):
            pltpu.make_async_copy(rows_v.at[b], acc.at[dst_v.at[b]],
                                  ssems[b]).wait()

        @pl.loop(0, NCH // IB)
        def _(ib):
            @pl.when(ib > 0)
            def _():
                _wait_scat(0)
                _wait_scat(1)

            base = t * NCH + ib * IB
            pltpu.sync_copy(src_hbm.at[pl.ds(base, IB)], src_v)
            pltpu.sync_copy(dst_hbm.at[pl.ds(base, IB)], dst_v)

            @pl.loop(0, IB, step=2)
            def _(j):
                @pl.when(j > 0)
                def _():
                    _wait_scat(0)
                    _wait_scat(1)

                d0 = pltpu.async_copy(g_hbm.at[src_v.at[j]], rows_v.at[0],
                                      gsems[0])
                d1 = pltpu.async_copy(g_hbm.at[src_v.at[j + 1]], rows_v.at[1],
                                      gsems[1])
                d0.wait()
                pltpu.async_copy(rows_v.at[0], acc.at[dst_v.at[j]], ssems[0],
                                 add=True)
                d1.wait()
                pltpu.async_copy(rows_v.at[1], acc.at[dst_v.at[j + 1]],
                                 ssems[1], add=True)

        _wait_scat(0)
        _wait_scat(1)
        plsc.subcore_barrier()
        pltpu.sync_copy(acc.at[pl.ds(s * RPT, RPT)], out_hbm.at[c, pl.ds(s * RPT, RPT)])

    return k(g, src2d, dst2d, zeros2d)


def _dinv_of(d0, d1):
    return lax.rsqrt(d0[...] + d1[...] + 1.0)[:, None]


def _tc_first(x_ref, w_ref, d0, d1, o_ref):
    dinv = _dinv_of(d0, d1)
    o_ref[...] = jnp.dot(x_ref[...], w_ref[...],
                         preferred_element_type=jnp.float32) * dinv


def _tc_mid(a0, a1, g_ref, b_ref, w_ref, d0, d1, o_ref):
    dinv = _dinv_of(d0, d1)
    z = (a0[...] + a1[...] + g_ref[...]) * dinv + b_ref[...][None, :]
    h = jnp.maximum(z, 0.0) + jnp.log1p(jnp.exp(-jnp.abs(z)))
    o_ref[...] = jnp.dot(h, w_ref[...],
                         preferred_element_type=jnp.float32) * dinv


def _tc_last(a0, a1, g_ref, b_ref, wf_ref, bf_ref, d0, d1, o_ref):
    dinv = _dinv_of(d0, d1)
    z = (a0[...] + a1[...] + g_ref[...]) * dinv + b_ref[...][None, :]
    logits = jnp.dot(z, wf_ref[...],
                     preferred_element_type=jnp.float32) + bf_ref[...][None, :]
    m = jnp.max(logits, axis=1, keepdims=True)
    e = jnp.exp(logits - m)
    o_ref[...] = e / jnp.sum(e, axis=1, keepdims=True)


def _row_spec():
    return pl.BlockSpec((BLK, F), lambda i: (i, 0))


def _full_spec(shape):
    nd = len(shape)
    return pl.BlockSpec(shape, lambda i: (0,) * nd)


def _vec_spec():
    return pl.BlockSpec((BLK,), lambda i: (i,))


def _tc_call(body, in_specs, args):
    return pl.pallas_call(
        body,
        grid=(NPAD // BLK,),
        in_specs=in_specs,
        out_specs=_row_spec(),
        out_shape=jax.ShapeDtypeStruct((NPAD, F), jnp.float32),
    )(*args)


def kernel(x, edge_index, W1, b1, W2, b2, Wf, bf):
    src = edge_index[0].astype(jnp.int32)
    dst = edge_index[1].astype(jnp.int32)
    pad = EPAD - E
    pad_dst = N + (jnp.arange(pad, dtype=jnp.int32) % (NPAD - N))

    def _chunked(a):
        a2 = a.reshape(NCH, TILES, CH).transpose(1, 0, 2)
        return a2.reshape(TILES * NCH, CH)

    src2d = _chunked(jnp.concatenate([src, jnp.full((pad,), N, jnp.int32)]))
    dst2d = _chunked(jnp.concatenate([dst, pad_dst]))
    xp = jnp.zeros((NPAD, F), x.dtype).at[:N].set(x)
    zeros1d = jnp.zeros((NPAD,), jnp.float32)
    zeros2d = jnp.zeros((NPAD, F), jnp.float32)

    deg = _sc_degree(dst2d, zeros1d)
    d0, d1 = deg[0], deg[1]

    g1 = _tc_call(
        _tc_first,
        [_row_spec(), _full_spec((F, F)), _vec_spec(), _vec_spec()],
        (xp, W1, d0, d1))
    acc1 = _sc_scatter_rows(g1, src2d, dst2d, zeros2d)

    g2 = _tc_call(
        _tc_mid,
        [_row_spec(), _row_spec(), _row_spec(), _full_spec((F,)),
         _full_spec((F, F)), _vec_spec(), _vec_spec()],
        (acc1[0], acc1[1], g1, b1, W2, d0, d1))
    acc2 = _sc_scatter_rows(g2, src2d, dst2d, zeros2d)

    wf_pad = jnp.zeros((F, F), jnp.float32).at[:, :7].set(Wf)
    bf_pad = jnp.full((F,), -1e30, jnp.float32).at[:7].set(bf)
    probs = _tc_call(
        _tc_last,
        [_row_spec(), _row_spec(), _row_spec(), _full_spec((F,)),
         _full_spec((F, F)), _full_spec((F,)), _vec_spec(), _vec_spec()],
        (acc2[0], acc2[1], g2, b2, wf_pad, bf_pad, d0, d1))
    return probs[:N, :7]


# speedup vs baseline: 1.4541x; 1.0000x over previous
import functools

import jax
import jax.numpy as jnp
from jax import lax
from jax.experimental import pallas as pl
from jax.experimental.pallas import tpu as pltpu
from jax.experimental.pallas import tpu_sc as plsc

N = 10000
F = 128
NPAD = 10240
E = 320000
CH = 128
NCH = 80
IB = 16
TILES = 32
EPAD = TILES * NCH * CH
RPT = NPAD // 16
BLK = 512

_MESH = dict(core_axis_name="c", subcore_axis_name="s")


def _sc_degree(dst2d, zeros1d):
    @functools.partial(
        pl.kernel,
        out_type=jax.ShapeDtypeStruct((2, NPAD), jnp.float32),
        mesh=plsc.VectorSubcoreMesh(**_MESH),
        scratch_types=[
            pltpu.VMEM((NCH, CH), jnp.int32),
            pltpu.VMEM((CH,), jnp.float32),
            pltpu.VMEM_SHARED((NPAD,), jnp.float32),
        ],
    )
    def k(dst_hbm, z_hbm, out_hbm, dst_v, ones_v, acc):
        c = lax.axis_index("c")
        s = lax.axis_index("s")
        t = c * 16 + s
        pltpu.sync_copy(dst_hbm.at[pl.ds(t * NCH, NCH)], dst_v)
        for i in range(CH // 16):
            ones_v[pl.ds(i * 16, 16)] = jnp.ones((16,), jnp.float32)
        pltpu.sync_copy(z_hbm.at[pl.ds(s * RPT, RPT)], acc.at[pl.ds(s * RPT, RPT)])
        plsc.subcore_barrier()

        @pl.loop(0, NCH)
        def _(j):
            pltpu.sync_copy(ones_v, acc.at[dst_v.at[j]], add=True)

        plsc.subcore_barrier()
        pltpu.sync_copy(acc.at[pl.ds(s * RPT, RPT)], out_hbm.at[c, pl.ds(s * RPT, RPT)])

    return k(dst2d, zeros1d)


def _sc_scatter_rows(g, src2d, dst2d, zeros2d):
    @functools.partial(
        pl.kernel,
        out_type=jax.ShapeDtypeStruct((2, NPAD, F), jnp.float32),
        mesh=plsc.VectorSubcoreMesh(**_MESH),
        scratch_types=[
            pltpu.VMEM((IB, CH), jnp.int32),
            pltpu.VMEM((IB, CH), jnp.int32),
            pltpu.VMEM((2, CH, F), jnp.float32),
            pltpu.VMEM_SHARED((NPAD, F), jnp.float32),
            [pltpu.SemaphoreType.DMA] * 2,
            [pltpu.SemaphoreType.DMA] * 2,
        ],
    )
    def k(g_hbm, src_hbm, dst_hbm, z_hbm, out_hbm, src_v, dst_v, rows_v, acc,
          gsems, ssems):
        c = lax.axis_index("c")
        s = lax.axis_index("s")
        t = c * 16 + s
        pltpu.sync_copy(z_hbm.at[pl.ds(s * RPT, RPT)], acc.at[pl.ds(s * RPT, RPT)])
        plsc.subcore_barrier()

        def _wait_scat(b):
            pltpu.make_async_copy(rows_v.at[b], acc.at[dst_v.at[b]],
                                  ssems[b]).wait()

        @pl.loop(0, NCH // IB)
        def _(ib):
            @pl.when(ib > 0)
            def _():
                _wait_scat(0)
                _wait_scat(1)

            base = t * NCH + ib * IB
            pltpu.sync_copy(src_hbm.at[pl.ds(base, IB)], src_v)
            pltpu.sync_copy(dst_hbm.at[pl.ds(base, IB)], dst_v)

            @pl.loop(0, IB, step=2, unroll=4)
            def _(j):
                @pl.when(j > 0)
                def _():
                    _wait_scat(0)
                    _wait_scat(1)

                d0 = pltpu.async_copy(g_hbm.at[src_v.at[j]], rows_v.at[0],
                                      gsems[0])
                d1 = pltpu.async_copy(g_hbm.at[src_v.at[j + 1]], rows_v.at[1],
                                      gsems[1])
                d0.wait()
                pltpu.async_copy(rows_v.at[0], acc.at[dst_v.at[j]], ssems[0],
                                 add=True)
                d1.wait()
                pltpu.async_copy(rows_v.at[1], acc.at[dst_v.at[j + 1]],
                                 ssems[1], add=True)

        _wait_scat(0)
        _wait_scat(1)
        plsc.subcore_barrier()
        pltpu.sync_copy(acc.at[pl.ds(s * RPT, RPT)], out_hbm.at[c, pl.ds(s * RPT, RPT)])

    return k(g, src2d, dst2d, zeros2d)


def _dinv_of(d0, d1):
    return lax.rsqrt(d0[...] + d1[...] + 1.0)[:, None]


def _tc_first(x_ref, w_ref, d0, d1, o_ref):
    dinv = _dinv_of(d0, d1)
    o_ref[...] = jnp.dot(x_ref[...], w_ref[...],
                         preferred_element_type=jnp.float32) * dinv


def _tc_mid(a0, a1, g_ref, b_ref, w_ref, d0, d1, o_ref):
    dinv = _dinv_of(d0, d1)
    z = (a0[...] + a1[...] + g_ref[...]) * dinv + b_ref[...][None, :]
    h = jnp.maximum(z, 0.0) + jnp.log1p(jnp.exp(-jnp.abs(z)))
    o_ref[...] = jnp.dot(h, w_ref[...],
                         preferred_element_type=jnp.float32) * dinv


def _tc_last(a0, a1, g_ref, b_ref, wf_ref, bf_ref, d0, d1, o_ref):
    dinv = _dinv_of(d0, d1)
    z = (a0[...] + a1[...] + g_ref[...]) * dinv + b_ref[...][None, :]
    logits = jnp.dot(z, wf_ref[...],
                     preferred_element_type=jnp.float32) + bf_ref[...][None, :]
    m = jnp.max(logits, axis=1, keepdims=True)
    e = jnp.exp(logits - m)
    o_ref[...] = e / jnp.sum(e, axis=1, keepdims=True)


def _row_spec():
    return pl.BlockSpec((BLK, F), lambda i: (i, 0))


def _full_spec(shape):
    nd = len(shape)
    return pl.BlockSpec(shape, lambda i: (0,) * nd)


def _vec_spec():
    return pl.BlockSpec((BLK,), lambda i: (i,))


def _tc_call(body, in_specs, args):
    return pl.pallas_call(
        body,
        grid=(NPAD // BLK,),
        in_specs=in_specs,
        out_specs=_row_spec(),
        out_shape=jax.ShapeDtypeStruct((NPAD, F), jnp.float32),
    )(*args)


def kernel(x, edge_index, W1, b1, W2, b2, Wf, bf):
    src = edge_index[0].astype(jnp.int32)
    dst = edge_index[1].astype(jnp.int32)
    pad = EPAD - E
    pad_dst = N + (jnp.arange(pad, dtype=jnp.int32) % (NPAD - N))

    def _chunked(a):
        a2 = a.reshape(NCH, TILES, CH).transpose(1, 0, 2)
        return a2.reshape(TILES * NCH, CH)

    src2d = _chunked(jnp.concatenate([src, jnp.full((pad,), N, jnp.int32)]))
    dst2d = _chunked(jnp.concatenate([dst, pad_dst]))
    xp = jnp.zeros((NPAD, F), x.dtype).at[:N].set(x)
    zeros1d = jnp.zeros((NPAD,), jnp.float32)
    zeros2d = jnp.zeros((NPAD, F), jnp.float32)

    deg = _sc_degree(dst2d, zeros1d)
    d0, d1 = deg[0], deg[1]

    g1 = _tc_call(
        _tc_first,
        [_row_spec(), _full_spec((F, F)), _vec_spec(), _vec_spec()],
        (xp, W1, d0, d1))
    acc1 = _sc_scatter_rows(g1, src2d, dst2d, zeros2d)

    g2 = _tc_call(
        _tc_mid,
        [_row_spec(), _row_spec(), _row_spec(), _full_spec((F,)),
         _full_spec((F, F)), _vec_spec(), _vec_spec()],
        (acc1[0], acc1[1], g1, b1, W2, d0, d1))
    acc2 = _sc_scatter_rows(g2, src2d, dst2d, zeros2d)

    wf_pad = jnp.zeros((F, F), jnp.float32).at[:, :7].set(Wf)
    bf_pad = jnp.full((F,), -1e30, jnp.float32).at[:7].set(bf)
    probs = _tc_call(
        _tc_last,
        [_row_spec(), _row_spec(), _row_spec(), _full_spec((F,)),
         _full_spec((F, F)), _full_spec((F,)), _vec_spec(), _vec_spec()],
        (acc2[0], acc2[1], g2, b2, wf_pad, bf_pad, d0, d1))
    return probs[:N, :7]


# speedup vs baseline: 1.4544x; 1.0002x over previous
import functools

import jax
import jax.numpy as jnp
from jax import lax
from jax.experimental import pallas as pl
from jax.experimental.pallas import tpu as pltpu
from jax.experimental.pallas import tpu_sc as plsc

N = 10000
F = 128
NPAD = 10240
E = 320000
CH = 128
NCH = 80
IB = 16
TILES = 32
EPAD = TILES * NCH * CH
RPT = NPAD // 16
BLK = 512

_MESH = dict(core_axis_name="c", subcore_axis_name="s")


def _sc_degree(dst2d, zeros1d):
    @functools.partial(
        pl.kernel,
        out_type=jax.ShapeDtypeStruct((2, NPAD), jnp.float32),
        mesh=plsc.VectorSubcoreMesh(**_MESH),
        scratch_types=[
            pltpu.VMEM((NCH, CH), jnp.int32),
            pltpu.VMEM((CH,), jnp.float32),
            pltpu.VMEM_SHARED((NPAD,), jnp.float32),
        ],
    )
    def k(dst_hbm, z_hbm, out_hbm, dst_v, ones_v, acc):
        c = lax.axis_index("c")
        s = lax.axis_index("s")
        t = c * 16 + s
        pltpu.sync_copy(dst_hbm.at[pl.ds(t * NCH, NCH)], dst_v)
        for i in range(CH // 16):
            ones_v[pl.ds(i * 16, 16)] = jnp.ones((16,), jnp.float32)
        pltpu.sync_copy(z_hbm.at[pl.ds(s * RPT, RPT)], acc.at[pl.ds(s * RPT, RPT)])
        plsc.subcore_barrier()

        @pl.loop(0, NCH)
        def _(j):
            pltpu.sync_copy(ones_v, acc.at[dst_v.at[j]], add=True)

        plsc.subcore_barrier()
        pltpu.sync_copy(acc.at[pl.ds(s * RPT, RPT)], out_hbm.at[c, pl.ds(s * RPT, RPT)])

    return k(dst2d, zeros1d)


def _sc_scatter_rows(g, src2d, dst2d, zeros2d):
    @functools.partial(
        pl.kernel,
        out_type=jax.ShapeDtypeStruct((2, NPAD, F), jnp.float32),
        mesh=plsc.VectorSubcoreMesh(**_MESH),
        scratch_types=[
            pltpu.VMEM((IB, CH), jnp.int32),
            pltpu.VMEM((IB, CH), jnp.int32),
            pltpu.VMEM((2, CH, F), jnp.float32),
            pltpu.VMEM_SHARED((NPAD, F), jnp.float32),
            [pltpu.SemaphoreType.DMA] * 2,
            [pltpu.SemaphoreType.DMA] * 2,
        ],
    )
    def k(g_hbm, src_hbm, dst_hbm, z_hbm, out_hbm, src_v, dst_v, rows_v, acc,
          gsems, ssems):
        c = lax.axis_index("c")
        s = lax.axis_index("s")
        t = c * 16 + s
        pltpu.sync_copy(z_hbm.at[pl.ds(s * RPT, RPT)], acc.at[pl.ds(s * RPT, RPT)])
        plsc.subcore_barrier()

        def _wait_scat(b):
            pltpu.make_async_copy(rows_v.at[b], acc.at[dst_v.at[b]],
                                  ssems[b]).wait()

        @pl.loop(0, NCH // IB)
        def _(ib):
            @pl.when(ib > 0)
            def _():
                _wait_scat(0)
                _wait_scat(1)

            base = t * NCH + ib * IB
            pltpu.sync_copy(src_hbm.at[pl.ds(base, IB)], src_v)
            pltpu.sync_copy(dst_hbm.at[pl.ds(base, IB)], dst_v)

            @pl.loop(0, IB, step=2)
            def _(j):
                @pl.when(j > 0)
                def _():
                    _wait_scat(0)
                    _wait_scat(1)

                d0 = pltpu.async_copy(g_hbm.at[src_v.at[j]], rows_v.at[0],
                                      gsems[0])
                d1 = pltpu.async_copy(g_hbm.at[src_v.at[j + 1]], rows_v.at[1],
                                      gsems[1])
                d0.wait()
                pltpu.async_copy(rows_v.at[0], acc.at[dst_v.at[j]], ssems[0],
                                 add=True)
                d1.wait()
                pltpu.async_copy(rows_v.at[1], acc.at[dst_v.at[j + 1]],
                                 ssems[1], add=True)

        _wait_scat(0)
        _wait_scat(1)
        plsc.subcore_barrier()
        pltpu.sync_copy(acc.at[pl.ds(s * RPT, RPT)], out_hbm.at[c, pl.ds(s * RPT, RPT)])

    return k(g, src2d, dst2d, zeros2d)


def _dinv_of(d0, d1):
    return lax.rsqrt(d0[...] + d1[...] + 1.0)[:, None]


def _tc_first(x_ref, w_ref, d0, d1, o_ref):
    dinv = _dinv_of(d0, d1)
    o_ref[...] = jnp.dot(x_ref[...], w_ref[...],
                         preferred_element_type=jnp.float32) * dinv


def _tc_mid(a0, a1, g_ref, b_ref, w_ref, d0, d1, o_ref):
    dinv = _dinv_of(d0, d1)
    z = (a0[...] + a1[...] + g_ref[...]) * dinv + b_ref[...][None, :]
    h = jnp.maximum(z, 0.0) + jnp.log1p(jnp.exp(-jnp.abs(z)))
    o_ref[...] = jnp.dot(h, w_ref[...],
                         preferred_element_type=jnp.float32) * dinv


def _tc_last(a0, a1, g_ref, b_ref, wf_ref, bf_ref, d0, d1, o_ref):
    dinv = _dinv_of(d0, d1)
    z = (a0[...] + a1[...] + g_ref[...]) * dinv + b_ref[...][None, :]
    logits = jnp.dot(z, wf_ref[...],
                     preferred_element_type=jnp.float32) + bf_ref[...][None, :]
    m = jnp.max(logits, axis=1, keepdims=True)
    e = jnp.exp(logits - m)
    o_ref[...] = e / jnp.sum(e, axis=1, keepdims=True)


def _row_spec():
    return pl.BlockSpec((BLK, F), lambda i: (i, 0))


def _full_spec(shape):
    nd = len(shape)
    return pl.BlockSpec(shape, lambda i: (0,) * nd)


def _vec_spec():
    return pl.BlockSpec((BLK,), lambda i: (i,))


def _tc_call(body, in_specs, args):
    return pl.pallas_call(
        body,
        grid=(NPAD // BLK,),
        in_specs=in_specs,
        out_specs=_row_spec(),
        out_shape=jax.ShapeDtypeStruct((NPAD, F), jnp.float32),
    )(*args)


def kernel(x, edge_index, W1, b1, W2, b2, Wf, bf):
    src = edge_index[0].astype(jnp.int32)
    dst = edge_index[1].astype(jnp.int32)
    pad = EPAD - E
    pad_dst = N + (jnp.arange(pad, dtype=jnp.int32) % (NPAD - N))

    def _chunked(a):
        a2 = a.reshape(NCH, TILES, CH).transpose(1, 0, 2)
        return a2.reshape(TILES * NCH, CH)

    src2d = _chunked(jnp.concatenate([src, jnp.full((pad,), N, jnp.int32)]))
    dst2d = _chunked(jnp.concatenate([dst, pad_dst]))
    xp = jnp.zeros((NPAD, F), x.dtype).at[:N].set(x)
    zeros1d = jnp.zeros((NPAD,), jnp.float32)
    zeros2d = jnp.zeros((NPAD, F), jnp.float32)

    deg = _sc_degree(dst2d, zeros1d)
    d0, d1 = deg[0], deg[1]

    g1 = _tc_call(
        _tc_first,
        [_row_spec(), _full_spec((F, F)), _vec_spec(), _vec_spec()],
        (xp, W1, d0, d1))
    acc1 = _sc_scatter_rows(g1, src2d, dst2d, zeros2d)

    g2 = _tc_call(
        _tc_mid,
        [_row_spec(), _row_spec(), _row_spec(), _full_spec((F,)),
         _full_spec((F, F)), _vec_spec(), _vec_spec()],
        (acc1[0], acc1[1], g1, b1, W2, d0, d1))
    acc2 = _sc_scatter_rows(g2, src2d, dst2d, zeros2d)

    wf_pad = jnp.zeros((F, F), jnp.float32).at[:, :7].set(Wf)
    bf_pad = jnp.full((F,), -1e30, jnp.float32).at[:7].set(bf)
    probs = _tc_call(
        _tc_last,
        [_row_spec(), _row_spec(), _row_spec(), _full_spec((F,)),
         _full_spec((F, F)), _full_spec((F,)), _vec_spec(), _vec_spec()],
        (acc2[0], acc2[1], g2, b2, wf_pad, bf_pad, d0, d1))
    return probs[:N, :7]
